# baseline (speedup 1.0000x reference)
import functools

import jax
import jax.numpy as jnp
from jax import lax
from jax.experimental import pallas as pl
from jax.experimental.pallas import tpu as pltpu
from jax.experimental.pallas import tpu_sc as plsc

_N = 10000
_E = 160000
_HC = 128
_NF = 128
_EC = 100
_IN = 5
_L = 4
_CUTOFF = 10.0

_K = 80
_NCHUNK = _E // _K
_NW = 32
_CPW = _NCHUNK // _NW
_XTRA = _NCHUNK - _CPW * _NW
_GROUPS = (_CPW + 1 + 4) // 6 + 1
_NP = 10240
_RPT = _NP // 16

_NB = 1000
_EB = 2000


def _ssp(x):
    return jax.nn.softplus(x) - jnp.log(2.0).astype(x.dtype)


def _mmt(a, w):
    return lax.dot_general(a, w, (((1,), (1,)), ((), ())),
                           preferred_element_type=jnp.float32)


def _pre_body(z5_ref, zh_ref, wemb_ref, bemb_ref, w1_ref, h_ref, xh_ref):
    h = _mmt(z5_ref[...], wemb_ref[...]) + bemb_ref[...] + zh_ref[...]
    h_ref[...] = h
    xh_ref[...] = _mmt(h, w1_ref[...])


def _pre(z5, zh, wemb, bemb, w10):
    grid = (_N // _NB,)
    return pl.pallas_call(
        _pre_body,
        grid=grid,
        in_specs=[
            pl.BlockSpec((_NB, _IN), lambda i: (i, 0)),
            pl.BlockSpec((_NB, _HC), lambda i: (i, 0)),
            pl.BlockSpec((_HC, _IN), lambda i: (0, 0)),
            pl.BlockSpec((1, _HC), lambda i: (0, 0)),
            pl.BlockSpec((_HC, _HC), lambda i: (0, 0)),
        ],
        out_specs=[
            pl.BlockSpec((_NB, _HC), lambda i: (i, 0)),
            pl.BlockSpec((_NB, _HC), lambda i: (i, 0)),
        ],
        out_shape=[
            jax.ShapeDtypeStruct((_N, _HC), jnp.float32),
            jax.ShapeDtypeStruct((_N, _HC), jnp.float32),
        ],
    )(z5, zh, wemb, bemb, w10)


def _cast_body(ea_ref, out_ref):
    out_ref[...] = ea_ref[...].astype(jnp.bfloat16)


def _cast(edge_attr):
    grid = (_E // 4000,)
    return pl.pallas_call(
        _cast_body,
        grid=grid,
        in_specs=[pl.BlockSpec((4000, _EC), lambda i: (i, 0))],
        out_specs=pl.BlockSpec((4000, _EC), lambda i: (i, 0)),
        out_shape=jax.ShapeDtypeStruct((_E, _EC), jnp.bfloat16),
    )(edge_attr)


def _filter_body(ea_ref, el_ref, wn1_ref, bn1_ref, wn2_ref, bn2_ref, out_ref):
    ea = ea_ref[...]
    c = (el_ref[...] <= _CUTOFF).astype(jnp.float32)
    t = _ssp(_mmt(ea, wn1_ref[...].astype(jnp.bfloat16)) + bn1_ref[...])
    w = _mmt(t.astype(jnp.bfloat16), wn2_ref[...].astype(jnp.bfloat16))
    out_ref[...] = ((w + bn2_ref[...]) * c).astype(jnp.bfloat16)


def _filter1(ea_bf, el2, wn1, bn1, wn2, bn2):
    grid = (_E // _EB,)
    return pl.pallas_call(
        _filter_body,
        grid=grid,
        in_specs=[
            pl.BlockSpec((_EB, _EC), lambda i: (i, 0)),
            pl.BlockSpec((_EB, 1), lambda i: (i, 0)),
            pl.BlockSpec((_NF, _EC), lambda i: (0, 0)),
            pl.BlockSpec((1, _NF), lambda i: (0, 0)),
            pl.BlockSpec((_NF, _NF), lambda i: (0, 0)),
            pl.BlockSpec((1, _NF), lambda i: (0, 0)),
        ],
        out_specs=pl.BlockSpec((_EB, _NF), lambda i: (i, 0)),
        out_shape=jax.ShapeDtypeStruct((_E, _NF), jnp.bfloat16),
    )(ea_bf, el2, wn1, bn1, wn2, bn2)


def _tail_body_xh(a0_ref, a1_ref, h_ref, w2_ref, b2_ref, wl_ref, bl_ref,
                  w1n_ref, hn_ref, xhn_ref):
    a = a0_ref[...] + a1_ref[...]
    t = _ssp(_mmt(a, w2_ref[...]) + b2_ref[...])
    hn = h_ref[...] + _mmt(t, wl_ref[...]) + bl_ref[...]
    hn_ref[...] = hn
    xhn_ref[...] = _mmt(hn, w1n_ref[...])


def _tail_body_last(a0_ref, a1_ref, h_ref, w2_ref, b2_ref, wl_ref, bl_ref,
                    hn_ref):
    a = a0_ref[...] + a1_ref[...]
    t = _ssp(_mmt(a, w2_ref[...]) + b2_ref[...])
    hn_ref[...] = h_ref[...] + _mmt(t, wl_ref[...]) + bl_ref[...]


def _tail(a0, a1, h, w2, b2, wl, bl, w1n):
    grid = (_N // _NB,)
    nblk = lambda i: (i, 0)
    wblk = lambda i: (0, 0)
    in_specs = [
        pl.BlockSpec((_NB, _HC), nblk),
        pl.BlockSpec((_NB, _HC), nblk),
        pl.BlockSpec((_NB, _HC), nblk),
        pl.BlockSpec((_HC, _HC), wblk),
        pl.BlockSpec((1, _HC), wblk),
        pl.BlockSpec((_HC, _HC), wblk),
        pl.BlockSpec((1, _HC), wblk),
    ]
    if w1n is None:
        return pl.pallas_call(
            _tail_body_last,
            grid=grid,
            in_specs=in_specs,
            out_specs=pl.BlockSpec((_NB, _HC), nblk),
            out_shape=jax.ShapeDtypeStruct((_N, _HC), jnp.float32),
        )(a0, a1, h, w2, b2, wl, bl)
    in_specs.append(pl.BlockSpec((_HC, _HC), wblk))
    return pl.pallas_call(
        _tail_body_xh,
        grid=grid,
        in_specs=in_specs,
        out_specs=[pl.BlockSpec((_NB, _HC), nblk),
                   pl.BlockSpec((_NB, _HC), nblk)],
        out_shape=[jax.ShapeDtypeStruct((_N, _HC), jnp.float32),
                   jax.ShapeDtypeStruct((_N, _HC), jnp.float32)],
    )(a0, a1, h, w2, b2, wl, bl, w1n)


@functools.partial(
    pl.kernel,
    out_type=jax.ShapeDtypeStruct((2 * _NP, _HC), jnp.float32),
    mesh=plsc.VectorSubcoreMesh(core_axis_name="c", subcore_axis_name="s"),
    scratch_types=[
        pltpu.VMEM_SHARED((_NP, _HC), jnp.float32),
        pltpu.VMEM((_K,), jnp.int32),
        pltpu.VMEM((_K,), jnp.int32),
        pltpu.VMEM((_K,), jnp.int32),
        pltpu.VMEM((_K,), jnp.int32),
        pltpu.VMEM((_K,), jnp.int32),
        pltpu.VMEM((_K,), jnp.int32),
        pltpu.VMEM((_K, _HC), jnp.float32),
        pltpu.VMEM((_K, _HC), jnp.float32),
        pltpu.VMEM((_K, _HC), jnp.bfloat16),
        pltpu.VMEM((_K, _HC), jnp.bfloat16),
        pltpu.SemaphoreType.DMA((3,)),
        pltpu.SemaphoreType.DMA((2,)),
        pltpu.SemaphoreType.DMA((2,)),
        pltpu.SemaphoreType.DMA((2,)),
    ],
)
def _sc_msg(xh_hbm, we_hbm, src_hbm, dst_hbm, out_hbm,
            agg_sh, sv0, sv1, sv2, dv0, dv1, dv2, rw0, rw1, wv0, wv1,
            isem, gsem, wsem, ssem):
    cid = lax.axis_index("c")
    sid = lax.axis_index("s")
    wid = sid * 2 + cid
    srcs = (sv0, sv1, sv2)
    dsts = (dv0, dv1, dv2)
    rows = (rw0, rw1)
    wevs = (wv0, wv1)
    msgs = rows
    start = wid * _CPW + jnp.minimum(wid, _XTRA)
    count = _CPW + (wid < _XTRA).astype(jnp.int32)

    def zb(i, carry):
        rw0[i, :] = jnp.zeros((_HC,), jnp.float32)
        return carry
    lax.fori_loop(0, _K, zb, 0)

    def za(k, carry):
        pltpu.sync_copy(rw0, agg_sh.at[pl.ds(sid * _RPT + k * _K, _K), :])
        return carry
    lax.fori_loop(0, _RPT // _K, za, 0)
    plsc.subcore_barrier()

    def idx_load(step, slot):
        base = (start + step) * _K
        pltpu.async_copy(src_hbm.at[pl.ds(base, _K)], srcs[slot], isem.at[slot])
        pltpu.async_copy(dst_hbm.at[pl.ds(base, _K)], dsts[slot], isem.at[slot])

    def idx_wait(slot):
        pltpu.make_async_copy(src_hbm.at[pl.ds(0, _K)], srcs[slot],
                              isem.at[slot]).wait()
        pltpu.make_async_copy(dst_hbm.at[pl.ds(0, _K)], dsts[slot],
                              isem.at[slot]).wait()

    def fetch(step, slot3, slot2):
        pltpu.async_copy(xh_hbm.at[srcs[slot3]], rows[slot2], gsem.at[slot2])
        base = (start + step) * _K
        pltpu.async_copy(we_hbm.at[pl.ds(base, _K), :], wevs[slot2],
                         wsem.at[slot2])

    pltpu.sync_copy(src_hbm.at[pl.ds(start * _K, _K)], sv0)
    pltpu.sync_copy(dst_hbm.at[pl.ds(start * _K, _K)], dv0)
    fetch(0, 0, 0)
    idx_load(1, 1)

    def group(g, carry):
        for b in range(6):
            t = g * 6 + b
            b2, nb2 = b % 2, (b + 1) % 2
            b3, nb3, pb3 = b % 3, (b + 1) % 3, (b + 2) % 3

            @pl.when((t >= 1) & (t - 1 < count))
            def _():
                pltpu.make_async_copy(msgs[nb2], agg_sh.at[dsts[pb3]],
                                      ssem.at[nb2]).wait()

            @pl.when(t + 1 < count)
            def _():
                idx_wait(nb3)
                fetch(t + 1, nb3, nb2)

            @pl.when(t + 2 < count)
            def _():
                idx_load(t + 2, pb3)

            @pl.when(t < count)
            def _():
                pltpu.make_async_copy(xh_hbm.at[srcs[b3]], rows[b2],
                                      gsem.at[b2]).wait()
                pltpu.make_async_copy(we_hbm.at[pl.ds(0, _K), :], wevs[b2],
                                      wsem.at[b2]).wait()
                rw, wv, mg = rows[b2], wevs[b2], msgs[b2]

                def mul(i, c2):
                    s = pl.ds(pl.multiple_of(4 * i, 4), 4)
                    mg[s, :] = rw[s, :] * wv[s, :].astype(jnp.float32)
                    return c2
                lax.fori_loop(0, _K // 4, mul, 0)
                pltpu.async_copy(mg, agg_sh.at[dsts[b3]], ssem.at[b2],
                                 add=True)
        return carry
    lax.fori_loop(0, _GROUPS, group, 0)

    plsc.subcore_barrier()
    pltpu.sync_copy(agg_sh.at[pl.ds(sid * _RPT, _RPT), :],
                    out_hbm.at[pl.ds(cid * _NP + sid * _RPT, _RPT), :])


def kernel(z, p_ctx, edge_index, edge_length, edge_attr,
           Wemb, bemb, W1, W2, b2, Wn1, bn1, Wn2, bn2, Wl, bl):
    del p_ctx
    z5 = z[:, :_IN]
    zh = z[:, _IN:]
    src = edge_index[0]
    dst = edge_index[1]
    el2 = edge_length.reshape(_E, 1)

    h, xh = _pre(z5, zh, Wemb, bemb.reshape(1, _HC), W1[0])
    ea_bf = _cast(edge_attr)
    we_all = [_filter1(ea_bf, el2, Wn1[i], bn1[i].reshape(1, _NF),
                       Wn2[i], bn2[i].reshape(1, _NF)) for i in range(_L)]

    for i in range(_L):
        aggp = _sc_msg(xh, we_all[i], src, dst)
        w1n = W1[i + 1] if i + 1 < _L else None
        res = _tail(aggp[:_N], aggp[_NP:_NP + _N], h,
                    W2[i], b2[i].reshape(1, _HC),
                    Wl[i], bl[i].reshape(1, _HC), w1n)
        if w1n is None:
            h = res
        else:
            h, xh = res
    return h


# speedup vs baseline: 1.0318x; 1.0318x over previous
import functools

import jax
import jax.numpy as jnp
from jax import lax
from jax.experimental import pallas as pl
from jax.experimental.pallas import tpu as pltpu
from jax.experimental.pallas import tpu_sc as plsc

_N = 10000
_E = 160000
_HC = 128
_NF = 128
_EC = 100
_IN = 5
_L = 4
_CUTOFF = 10.0

_K = 80
_NCHUNK = _E // _K
_NW = 32
_CPW = _NCHUNK // _NW
_XTRA = _NCHUNK - _CPW * _NW
_GROUPS = (_CPW + 1 + 4) // 6 + 1
_NP = 10240
_RPT = _NP // 16

_NB = 1000
_EB = 2000


def _ssp(x):
    return jax.nn.softplus(x) - jnp.log(2.0).astype(x.dtype)


def _mmt(a, w):
    return lax.dot_general(a, w, (((1,), (1,)), ((), ())),
                           preferred_element_type=jnp.float32)


def _pre_body(z5_ref, zh_ref, wemb_ref, bemb_ref, w1_ref, h_ref, xh_ref):
    h = _mmt(z5_ref[...], wemb_ref[...]) + bemb_ref[...] + zh_ref[...]
    h_ref[...] = h
    xh_ref[...] = _mmt(h.astype(jnp.bfloat16),
                       w1_ref[...].astype(jnp.bfloat16))


def _pre(z5, zh, wemb, bemb, w10):
    grid = (_N // _NB,)
    return pl.pallas_call(
        _pre_body,
        grid=grid,
        in_specs=[
            pl.BlockSpec((_NB, _IN), lambda i: (i, 0)),
            pl.BlockSpec((_NB, _HC), lambda i: (i, 0)),
            pl.BlockSpec((_HC, _IN), lambda i: (0, 0)),
            pl.BlockSpec((1, _HC), lambda i: (0, 0)),
            pl.BlockSpec((_HC, _HC), lambda i: (0, 0)),
        ],
        out_specs=[
            pl.BlockSpec((_NB, _HC), lambda i: (i, 0)),
            pl.BlockSpec((_NB, _HC), lambda i: (i, 0)),
        ],
        out_shape=[
            jax.ShapeDtypeStruct((_N, _HC), jnp.float32),
            jax.ShapeDtypeStruct((_N, _HC), jnp.float32),
        ],
    )(z5, zh, wemb, bemb, w10)


def _we_from(ea, el, wn1_ref, bn1_ref, wn2_ref, bn2_ref):
    c = (el <= _CUTOFF).astype(jnp.float32)
    t = _ssp(_mmt(ea, wn1_ref[...].astype(jnp.bfloat16)) + bn1_ref[...])
    w = _mmt(t.astype(jnp.bfloat16), wn2_ref[...].astype(jnp.bfloat16))
    return ((w + bn2_ref[...]) * c).astype(jnp.bfloat16)


def _filter0_body(ea_ref, el_ref, wn1_ref, bn1_ref, wn2_ref, bn2_ref,
                  out_ref, eab_ref):
    ea = ea_ref[...].astype(jnp.bfloat16)
    eab_ref[...] = ea
    out_ref[...] = _we_from(ea, el_ref[...], wn1_ref, bn1_ref, wn2_ref,
                            bn2_ref)


def _filter0(edge_attr, el2, wn1, bn1, wn2, bn2):
    grid = (_E // _EB,)
    return pl.pallas_call(
        _filter0_body,
        grid=grid,
        in_specs=[
            pl.BlockSpec((_EB, _EC), lambda i: (i, 0)),
            pl.BlockSpec((_EB, 1), lambda i: (i, 0)),
            pl.BlockSpec((_NF, _EC), lambda i: (0, 0)),
            pl.BlockSpec((1, _NF), lambda i: (0, 0)),
            pl.BlockSpec((_NF, _NF), lambda i: (0, 0)),
            pl.BlockSpec((1, _NF), lambda i: (0, 0)),
        ],
        out_specs=[pl.BlockSpec((_EB, _NF), lambda i: (i, 0)),
                   pl.BlockSpec((_EB, _EC), lambda i: (i, 0))],
        out_shape=[jax.ShapeDtypeStruct((_E, _NF), jnp.bfloat16),
                   jax.ShapeDtypeStruct((_E, _EC), jnp.bfloat16)],
    )(edge_attr, el2, wn1, bn1, wn2, bn2)


def _filter_body(ea_ref, el_ref, wn1_ref, bn1_ref, wn2_ref, bn2_ref, out_ref):
    out_ref[...] = _we_from(ea_ref[...], el_ref[...], wn1_ref, bn1_ref,
                            wn2_ref, bn2_ref)


def _filter1(ea_bf, el2, wn1, bn1, wn2, bn2):
    grid = (_E // _EB,)
    return pl.pallas_call(
        _filter_body,
        grid=grid,
        in_specs=[
            pl.BlockSpec((_EB, _EC), lambda i: (i, 0)),
            pl.BlockSpec((_EB, 1), lambda i: (i, 0)),
            pl.BlockSpec((_NF, _EC), lambda i: (0, 0)),
            pl.BlockSpec((1, _NF), lambda i: (0, 0)),
            pl.BlockSpec((_NF, _NF), lambda i: (0, 0)),
            pl.BlockSpec((1, _NF), lambda i: (0, 0)),
        ],
        out_specs=pl.BlockSpec((_EB, _NF), lambda i: (i, 0)),
        out_shape=jax.ShapeDtypeStruct((_E, _NF), jnp.bfloat16),
    )(ea_bf, el2, wn1, bn1, wn2, bn2)


def _tail_body_xh(a0_ref, a1_ref, h_ref, w2_ref, b2_ref, wl_ref, bl_ref,
                  w1n_ref, hn_ref, xhn_ref):
    a = (a0_ref[...] + a1_ref[...]).astype(jnp.bfloat16)
    t = _ssp(_mmt(a, w2_ref[...].astype(jnp.bfloat16)) + b2_ref[...])
    hn = (h_ref[...] + _mmt(t.astype(jnp.bfloat16),
                            wl_ref[...].astype(jnp.bfloat16)) + bl_ref[...])
    hn_ref[...] = hn
    xhn_ref[...] = _mmt(hn.astype(jnp.bfloat16),
                        w1n_ref[...].astype(jnp.bfloat16))


def _tail_body_last(a0_ref, a1_ref, h_ref, w2_ref, b2_ref, wl_ref, bl_ref,
                    hn_ref):
    a = (a0_ref[...] + a1_ref[...]).astype(jnp.bfloat16)
    t = _ssp(_mmt(a, w2_ref[...].astype(jnp.bfloat16)) + b2_ref[...])
    hn_ref[...] = (h_ref[...] + _mmt(t.astype(jnp.bfloat16),
                                     wl_ref[...].astype(jnp.bfloat16))
                   + bl_ref[...])


def _tail(a0, a1, h, w2, b2, wl, bl, w1n):
    grid = (_N // _NB,)
    nblk = lambda i: (i, 0)
    wblk = lambda i: (0, 0)
    in_specs = [
        pl.BlockSpec((_NB, _HC), nblk),
        pl.BlockSpec((_NB, _HC), nblk),
        pl.BlockSpec((_NB, _HC), nblk),
        pl.BlockSpec((_HC, _HC), wblk),
        pl.BlockSpec((1, _HC), wblk),
        pl.BlockSpec((_HC, _HC), wblk),
        pl.BlockSpec((1, _HC), wblk),
    ]
    if w1n is None:
        return pl.pallas_call(
            _tail_body_last,
            grid=grid,
            in_specs=in_specs,
            out_specs=pl.BlockSpec((_NB, _HC), nblk),
            out_shape=jax.ShapeDtypeStruct((_N, _HC), jnp.float32),
        )(a0, a1, h, w2, b2, wl, bl)
    in_specs.append(pl.BlockSpec((_HC, _HC), wblk))
    return pl.pallas_call(
        _tail_body_xh,
        grid=grid,
        in_specs=in_specs,
        out_specs=[pl.BlockSpec((_NB, _HC), nblk),
                   pl.BlockSpec((_NB, _HC), nblk)],
        out_shape=[jax.ShapeDtypeStruct((_N, _HC), jnp.float32),
                   jax.ShapeDtypeStruct((_N, _HC), jnp.float32)],
    )(a0, a1, h, w2, b2, wl, bl, w1n)


@functools.partial(
    pl.kernel,
    out_type=jax.ShapeDtypeStruct((2 * _NP, _HC), jnp.float32),
    mesh=plsc.VectorSubcoreMesh(core_axis_name="c", subcore_axis_name="s"),
    scratch_types=[
        pltpu.VMEM_SHARED((_NP, _HC), jnp.float32),
        pltpu.VMEM((_K,), jnp.int32),
        pltpu.VMEM((_K,), jnp.int32),
        pltpu.VMEM((_K,), jnp.int32),
        pltpu.VMEM((_K,), jnp.int32),
        pltpu.VMEM((_K,), jnp.int32),
        pltpu.VMEM((_K,), jnp.int32),
        pltpu.VMEM((_K, _HC), jnp.float32),
        pltpu.VMEM((_K, _HC), jnp.float32),
        pltpu.VMEM((_K, _HC), jnp.bfloat16),
        pltpu.VMEM((_K, _HC), jnp.bfloat16),
        pltpu.SemaphoreType.DMA((3,)),
        pltpu.SemaphoreType.DMA((2,)),
        pltpu.SemaphoreType.DMA((2,)),
        pltpu.SemaphoreType.DMA((2,)),
    ],
)
def _sc_msg(xh_hbm, we_hbm, src_hbm, dst_hbm, out_hbm,
            agg_sh, sv0, sv1, sv2, dv0, dv1, dv2, rw0, rw1, wv0, wv1,
            isem, gsem, wsem, ssem):
    cid = lax.axis_index("c")
    sid = lax.axis_index("s")
    wid = sid * 2 + cid
    srcs = (sv0, sv1, sv2)
    dsts = (dv0, dv1, dv2)
    rows = (rw0, rw1)
    wevs = (wv0, wv1)
    msgs = rows
    start = wid * _CPW + jnp.minimum(wid, _XTRA)
    count = _CPW + (wid < _XTRA).astype(jnp.int32)

    def zb(i, carry):
        rw0[i, :] = jnp.zeros((_HC,), jnp.float32)
        return carry
    lax.fori_loop(0, _K, zb, 0)

    def za(k, carry):
        pltpu.sync_copy(rw0, agg_sh.at[pl.ds(sid * _RPT + k * _K, _K), :])
        return carry
    lax.fori_loop(0, _RPT // _K, za, 0)
    plsc.subcore_barrier()

    def idx_load(step, slot):
        base = (start + step) * _K
        pltpu.async_copy(src_hbm.at[pl.ds(base, _K)], srcs[slot], isem.at[slot])
        pltpu.async_copy(dst_hbm.at[pl.ds(base, _K)], dsts[slot], isem.at[slot])

    def idx_wait(slot):
        pltpu.make_async_copy(src_hbm.at[pl.ds(0, _K)], srcs[slot],
                              isem.at[slot]).wait()
        pltpu.make_async_copy(dst_hbm.at[pl.ds(0, _K)], dsts[slot],
                              isem.at[slot]).wait()

    def fetch(step, slot3, slot2):
        pltpu.async_copy(xh_hbm.at[srcs[slot3]], rows[slot2], gsem.at[slot2])
        base = (start + step) * _K
        pltpu.async_copy(we_hbm.at[pl.ds(base, _K), :], wevs[slot2],
                         wsem.at[slot2])

    pltpu.sync_copy(src_hbm.at[pl.ds(start * _K, _K)], sv0)
    pltpu.sync_copy(dst_hbm.at[pl.ds(start * _K, _K)], dv0)
    fetch(0, 0, 0)
    idx_load(1, 1)

    def group(g, carry):
        for b in range(6):
            t = g * 6 + b
            b2, nb2 = b % 2, (b + 1) % 2
            b3, nb3, pb3 = b % 3, (b + 1) % 3, (b + 2) % 3

            @pl.when((t >= 1) & (t - 1 < count))
            def _():
                pltpu.make_async_copy(msgs[nb2], agg_sh.at[dsts[pb3]],
                                      ssem.at[nb2]).wait()

            @pl.when(t + 1 < count)
            def _():
                idx_wait(nb3)
                fetch(t + 1, nb3, nb2)

            @pl.when(t + 2 < count)
            def _():
                idx_load(t + 2, pb3)

            @pl.when(t < count)
            def _():
                pltpu.make_async_copy(xh_hbm.at[srcs[b3]], rows[b2],
                                      gsem.at[b2]).wait()
                pltpu.make_async_copy(we_hbm.at[pl.ds(0, _K), :], wevs[b2],
                                      wsem.at[b2]).wait()
                rw, wv, mg = rows[b2], wevs[b2], msgs[b2]

                def mul(i, c2):
                    s = pl.ds(pl.multiple_of(2 * i, 2), 2)
                    mg[s, :] = rw[s, :] * wv[s, :].astype(jnp.float32)
                    return c2
                lax.fori_loop(0, _K // 2, mul, 0)
                pltpu.async_copy(mg, agg_sh.at[dsts[b3]], ssem.at[b2],
                                 add=True)
        return carry
    lax.fori_loop(0, _GROUPS, group, 0)

    plsc.subcore_barrier()
    pltpu.sync_copy(agg_sh.at[pl.ds(sid * _RPT, _RPT), :],
                    out_hbm.at[pl.ds(cid * _NP + sid * _RPT, _RPT), :])


def kernel(z, p_ctx, edge_index, edge_length, edge_attr,
           Wemb, bemb, W1, W2, b2, Wn1, bn1, Wn2, bn2, Wl, bl):
    del p_ctx
    z5 = z[:, :_IN]
    zh = z[:, _IN:]
    src = edge_index[0]
    dst = edge_index[1]
    el2 = edge_length.reshape(_E, 1)

    h, xh = _pre(z5, zh, Wemb, bemb.reshape(1, _HC), W1[0])
    we0, ea_bf = _filter0(edge_attr, el2, Wn1[0], bn1[0].reshape(1, _NF),
                          Wn2[0], bn2[0].reshape(1, _NF))
    we_all = [we0] + [_filter1(ea_bf, el2, Wn1[i], bn1[i].reshape(1, _NF),
                               Wn2[i], bn2[i].reshape(1, _NF))
                      for i in range(1, _L)]

    for i in range(_L):
        aggp = _sc_msg(xh, we_all[i], src, dst)
        w1n = W1[i + 1] if i + 1 < _L else None
        res = _tail(aggp[:_N], aggp[_NP:_NP + _N], h,
                    W2[i], b2[i].reshape(1, _HC),
                    Wl[i], bl[i].reshape(1, _HC), w1n)
        if w1n is None:
            h = res
        else:
            h, xh = res
    return h


# speedup vs baseline: 1.1420x; 1.1068x over previous
import functools

import jax
import jax.numpy as jnp
from jax import lax
from jax.experimental import pallas as pl
from jax.experimental.pallas import tpu as pltpu
from jax.experimental.pallas import tpu_sc as plsc

_N = 10000
_E = 160000
_HC = 128
_NF = 128
_EC = 100
_IN = 5
_L = 4
_CUTOFF = 10.0

_K = 80
_NCHUNK = _E // _K
_NW = 32
_CPW = _NCHUNK // _NW
_XTRA = _NCHUNK - _CPW * _NW
_GROUPS = (_CPW + 1 + 4) // 6 + 1
_NP = 10240
_RPT = _NP // 16

_NB = 2000
_EB = 3200


def _ssp(x):
    return jax.nn.softplus(x) - jnp.log(2.0).astype(x.dtype)


def _mmt(a, w):
    return lax.dot_general(a, w, (((1,), (1,)), ((), ())),
                           preferred_element_type=jnp.float32)


def _pre_body(z5_ref, zh_ref, wemb_ref, bemb_ref, w1_ref, h_ref, xh_ref):
    h = _mmt(z5_ref[...], wemb_ref[...]) + bemb_ref[...] + zh_ref[...]
    h_ref[...] = h
    xh_ref[...] = _mmt(h.astype(jnp.bfloat16),
                       w1_ref[...].astype(jnp.bfloat16))


def _pre(z5, zh, wemb, bemb, w10):
    grid = (_N // _NB,)
    return pl.pallas_call(
        _pre_body,
        grid=grid,
        in_specs=[
            pl.BlockSpec((_NB, _IN), lambda i: (i, 0)),
            pl.BlockSpec((_NB, _HC), lambda i: (i, 0)),
            pl.BlockSpec((_HC, _IN), lambda i: (0, 0)),
            pl.BlockSpec((1, _HC), lambda i: (0, 0)),
            pl.BlockSpec((_HC, _HC), lambda i: (0, 0)),
        ],
        out_specs=[
            pl.BlockSpec((_NB, _HC), lambda i: (i, 0)),
            pl.BlockSpec((_NB, _HC), lambda i: (i, 0)),
        ],
        out_shape=[
            jax.ShapeDtypeStruct((_N, _HC), jnp.float32),
            jax.ShapeDtypeStruct((_N, _HC), jnp.float32),
        ],
    )(z5, zh, wemb, bemb, w10)


def _we_from(ea, el, wn1_ref, bn1_ref, wn2_ref, bn2_ref):
    c = (el <= _CUTOFF).astype(jnp.float32)
    t = _ssp(_mmt(ea, wn1_ref[...].astype(jnp.bfloat16)) + bn1_ref[...])
    w = _mmt(t.astype(jnp.bfloat16), wn2_ref[...].astype(jnp.bfloat16))
    return ((w + bn2_ref[...]) * c).astype(jnp.bfloat16)


def _filter0_body(ea_ref, el_ref, wn1_ref, bn1_ref, wn2_ref, bn2_ref,
                  out_ref, eab_ref):
    ea = ea_ref[...].astype(jnp.bfloat16)
    eab_ref[...] = ea
    out_ref[...] = _we_from(ea, el_ref[...], wn1_ref, bn1_ref, wn2_ref,
                            bn2_ref)


def _filter0(edge_attr, el2, wn1, bn1, wn2, bn2):
    grid = (_E // _EB,)
    return pl.pallas_call(
        _filter0_body,
        grid=grid,
        in_specs=[
            pl.BlockSpec((_EB, _EC), lambda i: (i, 0)),
            pl.BlockSpec((_EB, 1), lambda i: (i, 0)),
            pl.BlockSpec((_NF, _EC), lambda i: (0, 0)),
            pl.BlockSpec((1, _NF), lambda i: (0, 0)),
            pl.BlockSpec((_NF, _NF), lambda i: (0, 0)),
            pl.BlockSpec((1, _NF), lambda i: (0, 0)),
        ],
        out_specs=[pl.BlockSpec((_EB, _NF), lambda i: (i, 0)),
                   pl.BlockSpec((_EB, _EC), lambda i: (i, 0))],
        out_shape=[jax.ShapeDtypeStruct((_E, _NF), jnp.bfloat16),
                   jax.ShapeDtypeStruct((_E, _EC), jnp.bfloat16)],
    )(edge_attr, el2, wn1, bn1, wn2, bn2)


def _filter_body(ea_ref, el_ref, wn1_ref, bn1_ref, wn2_ref, bn2_ref, out_ref):
    out_ref[...] = _we_from(ea_ref[...], el_ref[...], wn1_ref, bn1_ref,
                            wn2_ref, bn2_ref)


def _filter1(ea_bf, el2, wn1, bn1, wn2, bn2):
    grid = (_E // _EB,)
    return pl.pallas_call(
        _filter_body,
        grid=grid,
        in_specs=[
            pl.BlockSpec((_EB, _EC), lambda i: (i, 0)),
            pl.BlockSpec((_EB, 1), lambda i: (i, 0)),
            pl.BlockSpec((_NF, _EC), lambda i: (0, 0)),
            pl.BlockSpec((1, _NF), lambda i: (0, 0)),
            pl.BlockSpec((_NF, _NF), lambda i: (0, 0)),
            pl.BlockSpec((1, _NF), lambda i: (0, 0)),
        ],
        out_specs=pl.BlockSpec((_EB, _NF), lambda i: (i, 0)),
        out_shape=jax.ShapeDtypeStruct((_E, _NF), jnp.bfloat16),
    )(ea_bf, el2, wn1, bn1, wn2, bn2)


def _tail_body_xh(a0_ref, a1_ref, h_ref, w2_ref, b2_ref, wl_ref, bl_ref,
                  w1n_ref, hn_ref, xhn_ref):
    a = (a0_ref[...] + a1_ref[...]).astype(jnp.bfloat16)
    t = _ssp(_mmt(a, w2_ref[...].astype(jnp.bfloat16)) + b2_ref[...])
    hn = (h_ref[...] + _mmt(t.astype(jnp.bfloat16),
                            wl_ref[...].astype(jnp.bfloat16)) + bl_ref[...])
    hn_ref[...] = hn
    xhn_ref[...] = _mmt(hn.astype(jnp.bfloat16),
                        w1n_ref[...].astype(jnp.bfloat16))


def _tail_body_last(a0_ref, a1_ref, h_ref, w2_ref, b2_ref, wl_ref, bl_ref,
                    hn_ref):
    a = (a0_ref[...] + a1_ref[...]).astype(jnp.bfloat16)
    t = _ssp(_mmt(a, w2_ref[...].astype(jnp.bfloat16)) + b2_ref[...])
    hn_ref[...] = (h_ref[...] + _mmt(t.astype(jnp.bfloat16),
                                     wl_ref[...].astype(jnp.bfloat16))
                   + bl_ref[...])


def _tail(a0, a1, h, w2, b2, wl, bl, w1n):
    grid = (_N // _NB,)
    nblk = lambda i: (i, 0)
    wblk = lambda i: (0, 0)
    in_specs = [
        pl.BlockSpec((_NB, _HC), nblk),
        pl.BlockSpec((_NB, _HC), nblk),
        pl.BlockSpec((_NB, _HC), nblk),
        pl.BlockSpec((_HC, _HC), wblk),
        pl.BlockSpec((1, _HC), wblk),
        pl.BlockSpec((_HC, _HC), wblk),
        pl.BlockSpec((1, _HC), wblk),
    ]
    if w1n is None:
        return pl.pallas_call(
            _tail_body_last,
            grid=grid,
            in_specs=in_specs,
            out_specs=pl.BlockSpec((_NB, _HC), nblk),
            out_shape=jax.ShapeDtypeStruct((_N, _HC), jnp.float32),
        )(a0, a1, h, w2, b2, wl, bl)
    in_specs.append(pl.BlockSpec((_HC, _HC), wblk))
    return pl.pallas_call(
        _tail_body_xh,
        grid=grid,
        in_specs=in_specs,
        out_specs=[pl.BlockSpec((_NB, _HC), nblk),
                   pl.BlockSpec((_NB, _HC), nblk)],
        out_shape=[jax.ShapeDtypeStruct((_N, _HC), jnp.float32),
                   jax.ShapeDtypeStruct((_N, _HC), jnp.float32)],
    )(a0, a1, h, w2, b2, wl, bl, w1n)


@functools.partial(
    pl.kernel,
    out_type=jax.ShapeDtypeStruct((2 * _NP, _HC), jnp.float32),
    mesh=plsc.VectorSubcoreMesh(core_axis_name="c", subcore_axis_name="s"),
    scratch_types=[
        pltpu.VMEM_SHARED((_NP, _HC), jnp.float32),
        pltpu.VMEM((_K,), jnp.int32),
        pltpu.VMEM((_K,), jnp.int32),
        pltpu.VMEM((_K,), jnp.int32),
        pltpu.VMEM((_K,), jnp.int32),
        pltpu.VMEM((_K,), jnp.int32),
        pltpu.VMEM((_K,), jnp.int32),
        pltpu.VMEM((_K, _HC), jnp.float32),
        pltpu.VMEM((_K, _HC), jnp.float32),
        pltpu.VMEM((_K, _HC), jnp.bfloat16),
        pltpu.VMEM((_K, _HC), jnp.bfloat16),
        pltpu.SemaphoreType.DMA((3,)),
        pltpu.SemaphoreType.DMA((2,)),
        pltpu.SemaphoreType.DMA((2,)),
        pltpu.SemaphoreType.DMA((2,)),
    ],
)
def _sc_msg(xh_hbm, we_hbm, src_hbm, dst_hbm, out_hbm,
            agg_sh, sv0, sv1, sv2, dv0, dv1, dv2, rw0, rw1, wv0, wv1,
            isem, gsem, wsem, ssem):
    cid = lax.axis_index("c")
    sid = lax.axis_index("s")
    wid = sid * 2 + cid
    srcs = (sv0, sv1, sv2)
    dsts = (dv0, dv1, dv2)
    rows = (rw0, rw1)
    wevs = (wv0, wv1)
    msgs = rows
    start = wid * _CPW + jnp.minimum(wid, _XTRA)
    count = _CPW + (wid < _XTRA).astype(jnp.int32)

    def zb(i, carry):
        rw0[i, :] = jnp.zeros((_HC,), jnp.float32)
        return carry
    lax.fori_loop(0, _K, zb, 0)

    def za(k, carry):
        pltpu.sync_copy(rw0, agg_sh.at[pl.ds(sid * _RPT + k * _K, _K), :])
        return carry
    lax.fori_loop(0, _RPT // _K, za, 0)
    plsc.subcore_barrier()

    def idx_load(step, slot):
        base = (start + step) * _K
        pltpu.async_copy(src_hbm.at[pl.ds(base, _K)], srcs[slot], isem.at[slot])
        pltpu.async_copy(dst_hbm.at[pl.ds(base, _K)], dsts[slot], isem.at[slot])

    def idx_wait(slot):
        pltpu.make_async_copy(src_hbm.at[pl.ds(0, _K)], srcs[slot],
                              isem.at[slot]).wait()
        pltpu.make_async_copy(dst_hbm.at[pl.ds(0, _K)], dsts[slot],
                              isem.at[slot]).wait()

    def fetch(step, slot3, slot2):
        pltpu.async_copy(xh_hbm.at[srcs[slot3]], rows[slot2], gsem.at[slot2])
        base = (start + step) * _K
        pltpu.async_copy(we_hbm.at[pl.ds(base, _K), :], wevs[slot2],
                         wsem.at[slot2])

    pltpu.sync_copy(src_hbm.at[pl.ds(start * _K, _K)], sv0)
    pltpu.sync_copy(dst_hbm.at[pl.ds(start * _K, _K)], dv0)
    fetch(0, 0, 0)
    idx_load(1, 1)

    def group(g, carry):
        for b in range(6):
            t = g * 6 + b
            b2, nb2 = b % 2, (b + 1) % 2
            b3, nb3, pb3 = b % 3, (b + 1) % 3, (b + 2) % 3

            @pl.when((t >= 1) & (t - 1 < count))
            def _():
                pltpu.make_async_copy(msgs[nb2], agg_sh.at[dsts[pb3]],
                                      ssem.at[nb2]).wait()

            @pl.when(t + 1 < count)
            def _():
                idx_wait(nb3)
                fetch(t + 1, nb3, nb2)

            @pl.when(t + 2 < count)
            def _():
                idx_load(t + 2, pb3)

            @pl.when(t < count)
            def _():
                pltpu.make_async_copy(xh_hbm.at[srcs[b3]], rows[b2],
                                      gsem.at[b2]).wait()
                pltpu.make_async_copy(we_hbm.at[pl.ds(0, _K), :], wevs[b2],
                                      wsem.at[b2]).wait()
                rw, wv, mg = rows[b2], wevs[b2], msgs[b2]

                def mul(i, c2):
                    s = pl.ds(pl.multiple_of(2 * i, 2), 2)
                    mg[s, :] = rw[s, :] * wv[s, :].astype(jnp.float32)
                    return c2
                lax.fori_loop(0, _K // 2, mul, 0)
                pltpu.async_copy(mg, agg_sh.at[dsts[b3]], ssem.at[b2],
                                 add=True)
        return carry
    lax.fori_loop(0, _GROUPS, group, 0)

    plsc.subcore_barrier()
    pltpu.sync_copy(agg_sh.at[pl.ds(sid * _RPT, _RPT), :],
                    out_hbm.at[pl.ds(cid * _NP + sid * _RPT, _RPT), :])


def kernel(z, p_ctx, edge_index, edge_length, edge_attr,
           Wemb, bemb, W1, W2, b2, Wn1, bn1, Wn2, bn2, Wl, bl):
    del p_ctx
    z5 = z[:, :_IN]
    zh = z[:, _IN:]
    src = edge_index[0]
    dst = edge_index[1]
    el2 = edge_length.reshape(_E, 1)

    h, xh = _pre(z5, zh, Wemb, bemb.reshape(1, _HC), W1[0])
    we0, ea_bf = _filter0(edge_attr, el2, Wn1[0], bn1[0].reshape(1, _NF),
                          Wn2[0], bn2[0].reshape(1, _NF))
    we_all = [we0] + [_filter1(ea_bf, el2, Wn1[i], bn1[i].reshape(1, _NF),
                               Wn2[i], bn2[i].reshape(1, _NF))
                      for i in range(1, _L)]

    for i in range(_L):
        aggp = _sc_msg(xh, we_all[i], src, dst)
        w1n = W1[i + 1] if i + 1 < _L else None
        res = _tail(aggp[:_N], aggp[_NP:_NP + _N], h,
                    W2[i], b2[i].reshape(1, _HC),
                    Wl[i], bl[i].reshape(1, _HC), w1n)
        if w1n is None:
            h = res
        else:
            h, xh = res
    return h


# speedup vs baseline: 1.1688x; 1.0235x over previous
import functools

import jax
import jax.numpy as jnp
from jax import lax
from jax.experimental import pallas as pl
from jax.experimental.pallas import tpu as pltpu
from jax.experimental.pallas import tpu_sc as plsc

_N = 10000
_E = 160000
_HC = 128
_NF = 128
_EC = 100
_IN = 5
_L = 4
_CUTOFF = 10.0

_K = 80
_NCHUNK = _E // _K
_NW = 32
_CPW = _NCHUNK // _NW
_XTRA = _NCHUNK - _CPW * _NW
_GROUPS = (_CPW + 1 + 4) // 6 + 1
_NP = 10240
_RPT = _NP // 16

_NB = 2000
_EB = 5000


def _ssp(x):
    return jax.nn.softplus(x) - jnp.log(2.0).astype(x.dtype)


def _mmt(a, w):
    return lax.dot_general(a, w, (((1,), (1,)), ((), ())),
                           preferred_element_type=jnp.float32)


def _pre_body(z5_ref, zh_ref, wemb_ref, bemb_ref, w1_ref, h_ref, xh_ref):
    h = _mmt(z5_ref[...], wemb_ref[...]) + bemb_ref[...] + zh_ref[...]
    h_ref[...] = h
    xh_ref[...] = _mmt(h.astype(jnp.bfloat16),
                       w1_ref[...].astype(jnp.bfloat16))


def _pre(z5, zh, wemb, bemb, w10):
    grid = (_N // _NB,)
    return pl.pallas_call(
        _pre_body,
        grid=grid,
        in_specs=[
            pl.BlockSpec((_NB, _IN), lambda i: (i, 0)),
            pl.BlockSpec((_NB, _HC), lambda i: (i, 0)),
            pl.BlockSpec((_HC, _IN), lambda i: (0, 0)),
            pl.BlockSpec((1, _HC), lambda i: (0, 0)),
            pl.BlockSpec((_HC, _HC), lambda i: (0, 0)),
        ],
        out_specs=[
            pl.BlockSpec((_NB, _HC), lambda i: (i, 0)),
            pl.BlockSpec((_NB, _HC), lambda i: (i, 0)),
        ],
        out_shape=[
            jax.ShapeDtypeStruct((_N, _HC), jnp.float32),
            jax.ShapeDtypeStruct((_N, _HC), jnp.float32),
        ],
    )(z5, zh, wemb, bemb, w10)


def _we_from(ea, el, wn1_ref, bn1_ref, wn2_ref, bn2_ref):
    c = (el <= _CUTOFF).astype(jnp.float32)
    t = _ssp(_mmt(ea, wn1_ref[...].astype(jnp.bfloat16)) + bn1_ref[...])
    w = _mmt(t.astype(jnp.bfloat16), wn2_ref[...].astype(jnp.bfloat16))
    return ((w + bn2_ref[...]) * c).astype(jnp.bfloat16)


def _filter0_body(ea_ref, el_ref, wn1_ref, bn1_ref, wn2_ref, bn2_ref,
                  out_ref, eab_ref):
    ea = ea_ref[...].astype(jnp.bfloat16)
    eab_ref[...] = ea
    out_ref[...] = _we_from(ea, el_ref[...], wn1_ref, bn1_ref, wn2_ref,
                            bn2_ref)


def _filter0(edge_attr, el2, wn1, bn1, wn2, bn2):
    grid = (_E // _EB,)
    return pl.pallas_call(
        _filter0_body,
        grid=grid,
        in_specs=[
            pl.BlockSpec((_EB, _EC), lambda i: (i, 0)),
            pl.BlockSpec((_EB, 1), lambda i: (i, 0)),
            pl.BlockSpec((_NF, _EC), lambda i: (0, 0)),
            pl.BlockSpec((1, _NF), lambda i: (0, 0)),
            pl.BlockSpec((_NF, _NF), lambda i: (0, 0)),
            pl.BlockSpec((1, _NF), lambda i: (0, 0)),
        ],
        out_specs=[pl.BlockSpec((_EB, _NF), lambda i: (i, 0)),
                   pl.BlockSpec((_EB, _EC), lambda i: (i, 0))],
        out_shape=[jax.ShapeDtypeStruct((_E, _NF), jnp.bfloat16),
                   jax.ShapeDtypeStruct((_E, _EC), jnp.bfloat16)],
    )(edge_attr, el2, wn1, bn1, wn2, bn2)


def _filter_body(ea_ref, el_ref, wn1_ref, bn1_ref, wn2_ref, bn2_ref, out_ref):
    out_ref[...] = _we_from(ea_ref[...], el_ref[...], wn1_ref, bn1_ref,
                            wn2_ref, bn2_ref)


def _filter1(ea_bf, el2, wn1, bn1, wn2, bn2):
    grid = (_E // _EB,)
    return pl.pallas_call(
        _filter_body,
        grid=grid,
        in_specs=[
            pl.BlockSpec((_EB, _EC), lambda i: (i, 0)),
            pl.BlockSpec((_EB, 1), lambda i: (i, 0)),
            pl.BlockSpec((_NF, _EC), lambda i: (0, 0)),
            pl.BlockSpec((1, _NF), lambda i: (0, 0)),
            pl.BlockSpec((_NF, _NF), lambda i: (0, 0)),
            pl.BlockSpec((1, _NF), lambda i: (0, 0)),
        ],
        out_specs=pl.BlockSpec((_EB, _NF), lambda i: (i, 0)),
        out_shape=jax.ShapeDtypeStruct((_E, _NF), jnp.bfloat16),
    )(ea_bf, el2, wn1, bn1, wn2, bn2)


def _tail_body_xh(a0_ref, a1_ref, h_ref, w2_ref, b2_ref, wl_ref, bl_ref,
                  w1n_ref, hn_ref, xhn_ref):
    a = (a0_ref[...] + a1_ref[...]).astype(jnp.bfloat16)
    t = _ssp(_mmt(a, w2_ref[...].astype(jnp.bfloat16)) + b2_ref[...])
    hn = (h_ref[...] + _mmt(t.astype(jnp.bfloat16),
                            wl_ref[...].astype(jnp.bfloat16)) + bl_ref[...])
    hn_ref[...] = hn
    xhn_ref[...] = _mmt(hn.astype(jnp.bfloat16),
                        w1n_ref[...].astype(jnp.bfloat16))


def _tail_body_last(a0_ref, a1_ref, h_ref, w2_ref, b2_ref, wl_ref, bl_ref,
                    hn_ref):
    a = (a0_ref[...] + a1_ref[...]).astype(jnp.bfloat16)
    t = _ssp(_mmt(a, w2_ref[...].astype(jnp.bfloat16)) + b2_ref[...])
    hn_ref[...] = (h_ref[...] + _mmt(t.astype(jnp.bfloat16),
                                     wl_ref[...].astype(jnp.bfloat16))
                   + bl_ref[...])


def _tail(a0, a1, h, w2, b2, wl, bl, w1n):
    grid = (_N // _NB,)
    nblk = lambda i: (i, 0)
    wblk = lambda i: (0, 0)
    in_specs = [
        pl.BlockSpec((_NB, _HC), nblk),
        pl.BlockSpec((_NB, _HC), nblk),
        pl.BlockSpec((_NB, _HC), nblk),
        pl.BlockSpec((_HC, _HC), wblk),
        pl.BlockSpec((1, _HC), wblk),
        pl.BlockSpec((_HC, _HC), wblk),
        pl.BlockSpec((1, _HC), wblk),
    ]
    if w1n is None:
        return pl.pallas_call(
            _tail_body_last,
            grid=grid,
            in_specs=in_specs,
            out_specs=pl.BlockSpec((_NB, _HC), nblk),
            out_shape=jax.ShapeDtypeStruct((_N, _HC), jnp.float32),
        )(a0, a1, h, w2, b2, wl, bl)
    in_specs.append(pl.BlockSpec((_HC, _HC), wblk))
    return pl.pallas_call(
        _tail_body_xh,
        grid=grid,
        in_specs=in_specs,
        out_specs=[pl.BlockSpec((_NB, _HC), nblk),
                   pl.BlockSpec((_NB, _HC), nblk)],
        out_shape=[jax.ShapeDtypeStruct((_N, _HC), jnp.float32),
                   jax.ShapeDtypeStruct((_N, _HC), jnp.float32)],
    )(a0, a1, h, w2, b2, wl, bl, w1n)


@functools.partial(
    pl.kernel,
    out_type=jax.ShapeDtypeStruct((2 * _NP, _HC), jnp.float32),
    mesh=plsc.VectorSubcoreMesh(core_axis_name="c", subcore_axis_name="s"),
    scratch_types=[
        pltpu.VMEM_SHARED((_NP, _HC), jnp.float32),
        pltpu.VMEM((_K,), jnp.int32),
        pltpu.VMEM((_K,), jnp.int32),
        pltpu.VMEM((_K,), jnp.int32),
        pltpu.VMEM((_K,), jnp.int32),
        pltpu.VMEM((_K,), jnp.int32),
        pltpu.VMEM((_K,), jnp.int32),
        pltpu.VMEM((_K, _HC), jnp.float32),
        pltpu.VMEM((_K, _HC), jnp.float32),
        pltpu.VMEM((_K, _HC), jnp.bfloat16),
        pltpu.VMEM((_K, _HC), jnp.bfloat16),
        pltpu.SemaphoreType.DMA((3,)),
        pltpu.SemaphoreType.DMA((2,)),
        pltpu.SemaphoreType.DMA((2,)),
        pltpu.SemaphoreType.DMA((2,)),
    ],
)
def _sc_msg(xh_hbm, we_hbm, src_hbm, dst_hbm, out_hbm,
            agg_sh, sv0, sv1, sv2, dv0, dv1, dv2, rw0, rw1, wv0, wv1,
            isem, gsem, wsem, ssem):
    cid = lax.axis_index("c")
    sid = lax.axis_index("s")
    wid = sid * 2 + cid
    srcs = (sv0, sv1, sv2)
    dsts = (dv0, dv1, dv2)
    rows = (rw0, rw1)
    wevs = (wv0, wv1)
    msgs = rows
    start = wid * _CPW + jnp.minimum(wid, _XTRA)
    count = _CPW + (wid < _XTRA).astype(jnp.int32)

    def zb(i, carry):
        rw0[i, :] = jnp.zeros((_HC,), jnp.float32)
        return carry
    lax.fori_loop(0, _K, zb, 0)

    def za(k, carry):
        pltpu.sync_copy(rw0, agg_sh.at[pl.ds(sid * _RPT + k * _K, _K), :])
        return carry
    lax.fori_loop(0, _RPT // _K, za, 0)
    plsc.subcore_barrier()

    def idx_load(step, slot):
        base = (start + step) * _K
        pltpu.async_copy(src_hbm.at[pl.ds(base, _K)], srcs[slot], isem.at[slot])
        pltpu.async_copy(dst_hbm.at[pl.ds(base, _K)], dsts[slot], isem.at[slot])

    def idx_wait(slot):
        pltpu.make_async_copy(src_hbm.at[pl.ds(0, _K)], srcs[slot],
                              isem.at[slot]).wait()
        pltpu.make_async_copy(dst_hbm.at[pl.ds(0, _K)], dsts[slot],
                              isem.at[slot]).wait()

    def fetch(step, slot3, slot2):
        pltpu.async_copy(xh_hbm.at[srcs[slot3]], rows[slot2], gsem.at[slot2])
        base = (start + step) * _K
        pltpu.async_copy(we_hbm.at[pl.ds(base, _K), :], wevs[slot2],
                         wsem.at[slot2])

    pltpu.sync_copy(src_hbm.at[pl.ds(start * _K, _K)], sv0)
    pltpu.sync_copy(dst_hbm.at[pl.ds(start * _K, _K)], dv0)
    fetch(0, 0, 0)
    idx_load(1, 1)

    def group(g, carry):
        for b in range(6):
            t = g * 6 + b
            b2, nb2 = b % 2, (b + 1) % 2
            b3, nb3, pb3 = b % 3, (b + 1) % 3, (b + 2) % 3

            @pl.when((t >= 1) & (t - 1 < count))
            def _():
                pltpu.make_async_copy(msgs[nb2], agg_sh.at[dsts[pb3]],
                                      ssem.at[nb2]).wait()

            @pl.when(t + 1 < count)
            def _():
                idx_wait(nb3)
                fetch(t + 1, nb3, nb2)

            @pl.when(t + 2 < count)
            def _():
                idx_load(t + 2, pb3)

            @pl.when(t < count)
            def _():
                pltpu.make_async_copy(xh_hbm.at[srcs[b3]], rows[b2],
                                      gsem.at[b2]).wait()
                pltpu.make_async_copy(we_hbm.at[pl.ds(0, _K), :], wevs[b2],
                                      wsem.at[b2]).wait()
                rw, wv, mg = rows[b2], wevs[b2], msgs[b2]

                def mul(i, c2):
                    s = pl.ds(pl.multiple_of(2 * i, 2), 2)
                    mg[s, :] = rw[s, :] * wv[s, :].astype(jnp.float32)
                    return c2
                lax.fori_loop(0, _K // 2, mul, 0)
                pltpu.async_copy(mg, agg_sh.at[dsts[b3]], ssem.at[b2],
                                 add=True)
        return carry
    lax.fori_loop(0, _GROUPS, group, 0)

    plsc.subcore_barrier()
    pltpu.sync_copy(agg_sh.at[pl.ds(sid * _RPT, _RPT), :],
                    out_hbm.at[pl.ds(cid * _NP + sid * _RPT, _RPT), :])


def kernel(z, p_ctx, edge_index, edge_length, edge_attr,
           Wemb, bemb, W1, W2, b2, Wn1, bn1, Wn2, bn2, Wl, bl):
    del p_ctx
    z5 = z[:, :_IN]
    zh = z[:, _IN:]
    src = edge_index[0]
    dst = edge_index[1]
    el2 = edge_length.reshape(_E, 1)

    h, xh = _pre(z5, zh, Wemb, bemb.reshape(1, _HC), W1[0])
    we0, ea_bf = _filter0(edge_attr, el2, Wn1[0], bn1[0].reshape(1, _NF),
                          Wn2[0], bn2[0].reshape(1, _NF))
    we_all = [we0] + [_filter1(ea_bf, el2, Wn1[i], bn1[i].reshape(1, _NF),
                               Wn2[i], bn2[i].reshape(1, _NF))
                      for i in range(1, _L)]

    for i in range(_L):
        aggp = _sc_msg(xh, we_all[i], src, dst)
        w1n = W1[i + 1] if i + 1 < _L else None
        res = _tail(aggp[:_N], aggp[_NP:_NP + _N], h,
                    W2[i], b2[i].reshape(1, _HC),
                    Wl[i], bl[i].reshape(1, _HC), w1n)
        if w1n is None:
            h = res
        else:
            h, xh = res
    return h
